# baseline (speedup 1.0000x reference)
import jax
import jax.numpy as jnp
from jax.experimental import pallas as pl

_N_CODES = 1024
_DIM = 64
_ROWS = 2048
_BLK = 2048
_GRID = _ROWS // _BLK
_CHUNK = 256
_NCHUNK = _N_CODES // _CHUNK
_COMMIT = 0.25
_HI = jax.lax.Precision.HIGHEST


def _vq_block(x_ref, cw_ref, q_ref, idx_ref, psum_ref):
    x = x_ref[...]
    eye = (jax.lax.broadcasted_iota(jnp.int32, (_DIM, _DIM), 0)
           == jax.lax.broadcasted_iota(jnp.int32, (_DIM, _DIM), 1)
           ).astype(jnp.float32)
    xt = jax.lax.dot_general(eye, x, (((1,), (1,)), ((), ())),
                             precision=_HI,
                             preferred_element_type=jnp.float32)

    iota_s = jax.lax.broadcasted_iota(jnp.int32, (_CHUNK, _BLK), 0)
    big = jnp.full((1, _BLK), jnp.inf, jnp.float32)
    bigi = jnp.full((1, _BLK), _N_CODES, jnp.int32)

    m1, i1, m2, i2 = big, bigi, big, bigi
    for j in range(_NCHUNK):
        cwj = cw_ref[pl.ds(j * _CHUNK, _CHUNK), :]
        ccj = jnp.sum(cwj * cwj, axis=1, keepdims=True)
        xc = jax.lax.dot_general(cwj, x, (((1,), (1,)), ((), ())),
                                 precision=_HI,
                                 preferred_element_type=jnp.float32)
        sj = ccj - 2.0 * xc
        gcol = iota_s + j * _CHUNK

        mj1 = jnp.min(sj, axis=0, keepdims=True)
        eq1 = sj == mj1
        ij1 = jnp.min(jnp.where(eq1, gcol, _N_CODES), axis=0, keepdims=True)
        sm = jnp.where(eq1, jnp.inf, sj)
        mj2 = jnp.min(sm, axis=0, keepdims=True)
        ij2 = jnp.min(jnp.where(sm == mj2, gcol, _N_CODES),
                      axis=0, keepdims=True)

        t = mj1 < m1
        lm = jnp.where(t, m1, mj1)
        li = jnp.where(t, i1, ij1)
        rm = jnp.where(t, mj2, m2)
        ri = jnp.where(t, ij2, i2)
        m1 = jnp.where(t, mj1, m1)
        i1 = jnp.where(t, ij1, i1)
        u = rm < lm
        m2 = jnp.where(u, rm, lm)
        i2 = jnp.where(u, ri, li)

    ct = jnp.zeros((_DIM, 2 * _BLK), jnp.float32)
    for j in range(_NCHUNK):
        cwj = cw_ref[pl.ds(j * _CHUNK, _CHUNK), :]
        gcol = iota_s + j * _CHUNK
        oh = jnp.concatenate([(gcol == i1).astype(jnp.float32),
                              (gcol == i2).astype(jnp.float32)],
                             axis=1)
        ct = ct + jax.lax.dot_general(cwj, oh, (((0,), (0,)), ((), ())),
                                      precision=_HI,
                                      preferred_element_type=jnp.float32)
    c1t = ct[:, :_BLK]
    c2t = ct[:, _BLK:]

    d1 = jnp.sum((xt - c1t) ** 2, axis=0, keepdims=True)
    d2 = jnp.sum((xt - c2t) ** 2, axis=0, keepdims=True)

    use2 = (d2 < d1) | ((d2 == d1) & (i2 < i1))
    idx = jnp.where(use2, i2, i1)
    qt = jnp.where(use2, c2t, c1t)

    q = jax.lax.dot_general(qt, eye, (((0,), (0,)), ((), ())),
                            precision=_HI,
                            preferred_element_type=jnp.float32)

    q_st = x + (q - x)
    q_ref[...] = q_st
    idx_ref[0, :, :] = idx
    e = (q_st - x) ** 2
    psum_ref[...] = jnp.sum(e).reshape(1, 1, 1)


def kernel(inputs, codewords):
    in_shape = inputs.shape
    x = inputs.reshape(_ROWS, _DIM)

    q_st, idx, psum = pl.pallas_call(
        _vq_block,
        grid=(_GRID,),
        in_specs=[
            pl.BlockSpec((_BLK, _DIM), lambda i: (i, 0)),
            pl.BlockSpec((_N_CODES, _DIM), lambda i: (0, 0)),
        ],
        out_specs=[
            pl.BlockSpec((_BLK, _DIM), lambda i: (i, 0)),
            pl.BlockSpec((1, 1, _BLK), lambda i: (i, 0, 0)),
            pl.BlockSpec((1, 1, 1), lambda i: (i, 0, 0)),
        ],
        out_shape=[
            jax.ShapeDtypeStruct((_ROWS, _DIM), jnp.float32),
            jax.ShapeDtypeStruct((_GRID, 1, _BLK), jnp.int32),
            jax.ShapeDtypeStruct((_GRID, 1, 1), jnp.float32),
        ],
    )(x, codewords)

    mean_e = jnp.sum(psum) / jnp.float32(_ROWS * _DIM)
    loss = mean_e + _COMMIT * mean_e
    return (q_st.reshape(in_shape),
            idx.reshape(in_shape[:-1]),
            loss)


# speedup vs baseline: 1.3326x; 1.3326x over previous
import jax
import jax.numpy as jnp
from jax.experimental import pallas as pl

_N_CODES = 1024
_DIM = 64
_ROWS = 2048
_BLK = 2048
_GRID = _ROWS // _BLK
_CHUNK = 128
_NCHUNK = _N_CODES // _CHUNK
_COMMIT = 0.25
_HI = jax.lax.Precision.HIGHEST


def _vq_block(x_ref, cw_ref, q_ref, idx_ref, psum_ref):
    x = x_ref[...]
    xt = jax.lax.transpose(x, (1, 0))

    cw = cw_ref[...]
    cw_hi = cw.astype(jnp.bfloat16)
    r1 = cw - cw_hi.astype(jnp.float32)
    cw_mid = r1.astype(jnp.bfloat16)
    cw_lo = (r1 - cw_mid.astype(jnp.float32)).astype(jnp.bfloat16)

    iota_s = jax.lax.broadcasted_iota(jnp.int32, (_CHUNK, _BLK), 0)
    big = jnp.full((1, _BLK), jnp.inf, jnp.float32)
    bigi = jnp.full((1, _BLK), _N_CODES, jnp.int32)

    m1, i1, m2, i2 = big, bigi, big, bigi
    for j in range(_NCHUNK):
        cwj = cw_ref[pl.ds(j * _CHUNK, _CHUNK), :]
        ccj = jnp.sum(cwj * cwj, axis=1, keepdims=True)
        xc = jax.lax.dot_general(cwj, x, (((1,), (1,)), ((), ())),
                                 precision=_HI,
                                 preferred_element_type=jnp.float32)
        sj = ccj - 2.0 * xc
        gcol = iota_s + j * _CHUNK

        mj1 = jnp.min(sj, axis=0, keepdims=True)
        eq1 = sj == mj1
        ij1 = jnp.min(jnp.where(eq1, gcol, _N_CODES), axis=0, keepdims=True)
        sm = jnp.where(eq1, jnp.inf, sj)
        mj2 = jnp.min(sm, axis=0, keepdims=True)
        ij2 = jnp.min(jnp.where(sm == mj2, gcol, _N_CODES),
                      axis=0, keepdims=True)

        t = mj1 < m1
        lm = jnp.where(t, m1, mj1)
        li = jnp.where(t, i1, ij1)
        rm = jnp.where(t, mj2, m2)
        ri = jnp.where(t, ij2, i2)
        m1 = jnp.where(t, mj1, m1)
        i1 = jnp.where(t, ij1, i1)
        u = rm < lm
        m2 = jnp.where(u, rm, lm)
        i2 = jnp.where(u, ri, li)

    c1t = jnp.zeros((_DIM, _BLK), jnp.float32)
    c2t = jnp.zeros((_DIM, _BLK), jnp.float32)
    for j in range(_NCHUNK):
        gcol = iota_s + j * _CHUNK
        oh1 = (gcol == i1).astype(jnp.bfloat16)
        oh2 = (gcol == i2).astype(jnp.bfloat16)
        for part in (cw_hi, cw_mid, cw_lo):
            pj = part[j * _CHUNK:(j + 1) * _CHUNK, :]
            c1t = c1t + jax.lax.dot_general(
                pj, oh1, (((0,), (0,)), ((), ())),
                preferred_element_type=jnp.float32)
            c2t = c2t + jax.lax.dot_general(
                pj, oh2, (((0,), (0,)), ((), ())),
                preferred_element_type=jnp.float32)

    d1 = jnp.sum((xt - c1t) ** 2, axis=0, keepdims=True)
    d2 = jnp.sum((xt - c2t) ** 2, axis=0, keepdims=True)

    use2 = (d2 < d1) | ((d2 == d1) & (i2 < i1))
    idx = jnp.where(use2, i2, i1)
    qt = jnp.where(use2, c2t, c1t)

    q = jax.lax.transpose(qt, (1, 0))

    q_st = x + (q - x)
    q_ref[...] = q_st
    idx_ref[0, :, :] = idx
    e = (q_st - x) ** 2
    psum_ref[...] = jnp.sum(e).reshape(1, 1, 1)


def kernel(inputs, codewords):
    in_shape = inputs.shape
    x = inputs.reshape(_ROWS, _DIM)

    q_st, idx, psum = pl.pallas_call(
        _vq_block,
        grid=(_GRID,),
        in_specs=[
            pl.BlockSpec((_BLK, _DIM), lambda i: (i, 0)),
            pl.BlockSpec((_N_CODES, _DIM), lambda i: (0, 0)),
        ],
        out_specs=[
            pl.BlockSpec((_BLK, _DIM), lambda i: (i, 0)),
            pl.BlockSpec((1, 1, _BLK), lambda i: (i, 0, 0)),
            pl.BlockSpec((1, 1, 1), lambda i: (i, 0, 0)),
        ],
        out_shape=[
            jax.ShapeDtypeStruct((_ROWS, _DIM), jnp.float32),
            jax.ShapeDtypeStruct((_GRID, 1, _BLK), jnp.int32),
            jax.ShapeDtypeStruct((_GRID, 1, 1), jnp.float32),
        ],
    )(x, codewords)

    mean_e = jnp.sum(psum) / jnp.float32(_ROWS * _DIM)
    loss = mean_e + _COMMIT * mean_e
    return (q_st.reshape(in_shape),
            idx.reshape(in_shape[:-1]),
            loss)


# speedup vs baseline: 1.4330x; 1.0753x over previous
import jax
import jax.numpy as jnp
from jax.experimental import pallas as pl

_N_CODES = 1024
_DIM = 64
_ROWS = 2048
_BLK = 2048
_GRID = _ROWS // _BLK
_CHUNK = 128
_NCHUNK = _N_CODES // _CHUNK
_COMMIT = 0.25
_HI = jax.lax.Precision.HIGHEST


def _vq_block(x_ref, cw_ref, q_ref, idx_ref, psum_ref):
    x = x_ref[...]
    xt = jax.lax.transpose(x, (1, 0))

    cw = cw_ref[...]
    cw_hi = cw.astype(jnp.bfloat16)
    r1 = cw - cw_hi.astype(jnp.float32)
    cw_mid = r1.astype(jnp.bfloat16)
    cw_lo = (r1 - cw_mid.astype(jnp.float32)).astype(jnp.bfloat16)
    x_hi = x.astype(jnp.bfloat16)
    x_lo = (x - x_hi.astype(jnp.float32)).astype(jnp.bfloat16)

    iota_s = jax.lax.broadcasted_iota(jnp.int32, (_CHUNK, _BLK), 0)
    big = jnp.full((1, _BLK), jnp.inf, jnp.float32)
    bigi = jnp.full((1, _BLK), _N_CODES, jnp.int32)

    m1, i1, m2, i2 = big, bigi, big, bigi
    for j in range(_NCHUNK):
        sl = slice(j * _CHUNK, (j + 1) * _CHUNK)
        cwj = cw[sl, :]
        ccj = jnp.sum(cwj * cwj, axis=1, keepdims=True)
        dims = (((1,), (1,)), ((), ()))
        xc = (jax.lax.dot_general(cw_hi[sl, :], x_hi, dims,
                                  preferred_element_type=jnp.float32)
              + jax.lax.dot_general(cw_mid[sl, :], x_hi, dims,
                                    preferred_element_type=jnp.float32)
              + jax.lax.dot_general(cw_hi[sl, :], x_lo, dims,
                                    preferred_element_type=jnp.float32))
        sj = ccj - 2.0 * xc
        gcol = iota_s + j * _CHUNK

        mj1 = jnp.min(sj, axis=0, keepdims=True)
        eq1 = sj == mj1
        ij1 = jnp.min(jnp.where(eq1, gcol, _N_CODES), axis=0, keepdims=True)
        sm = jnp.where(eq1, jnp.inf, sj)
        mj2 = jnp.min(sm, axis=0, keepdims=True)
        ij2 = jnp.min(jnp.where(sm == mj2, gcol, _N_CODES),
                      axis=0, keepdims=True)

        t = mj1 < m1
        lm = jnp.where(t, m1, mj1)
        li = jnp.where(t, i1, ij1)
        rm = jnp.where(t, mj2, m2)
        ri = jnp.where(t, ij2, i2)
        m1 = jnp.where(t, mj1, m1)
        i1 = jnp.where(t, ij1, i1)
        u = rm < lm
        m2 = jnp.where(u, rm, lm)
        i2 = jnp.where(u, ri, li)

    c1t = jnp.zeros((_DIM, _BLK), jnp.float32)
    c2t = jnp.zeros((_DIM, _BLK), jnp.float32)
    for j in range(_NCHUNK):
        gcol = iota_s + j * _CHUNK
        oh1 = (gcol == i1).astype(jnp.bfloat16)
        oh2 = (gcol == i2).astype(jnp.bfloat16)
        for part in (cw_hi, cw_mid, cw_lo):
            pj = part[j * _CHUNK:(j + 1) * _CHUNK, :]
            c1t = c1t + jax.lax.dot_general(
                pj, oh1, (((0,), (0,)), ((), ())),
                preferred_element_type=jnp.float32)
            c2t = c2t + jax.lax.dot_general(
                pj, oh2, (((0,), (0,)), ((), ())),
                preferred_element_type=jnp.float32)

    d1 = jnp.sum((xt - c1t) ** 2, axis=0, keepdims=True)
    d2 = jnp.sum((xt - c2t) ** 2, axis=0, keepdims=True)

    use2 = (d2 < d1) | ((d2 == d1) & (i2 < i1))
    idx = jnp.where(use2, i2, i1)
    qt = jnp.where(use2, c2t, c1t)

    q = jax.lax.transpose(qt, (1, 0))

    q_st = x + (q - x)
    q_ref[...] = q_st
    idx_ref[0, :, :] = idx
    e = (q_st - x) ** 2
    psum_ref[...] = jnp.sum(e).reshape(1, 1, 1)


def kernel(inputs, codewords):
    in_shape = inputs.shape
    x = inputs.reshape(_ROWS, _DIM)

    q_st, idx, psum = pl.pallas_call(
        _vq_block,
        grid=(_GRID,),
        in_specs=[
            pl.BlockSpec((_BLK, _DIM), lambda i: (i, 0)),
            pl.BlockSpec((_N_CODES, _DIM), lambda i: (0, 0)),
        ],
        out_specs=[
            pl.BlockSpec((_BLK, _DIM), lambda i: (i, 0)),
            pl.BlockSpec((1, 1, _BLK), lambda i: (i, 0, 0)),
            pl.BlockSpec((1, 1, 1), lambda i: (i, 0, 0)),
        ],
        out_shape=[
            jax.ShapeDtypeStruct((_ROWS, _DIM), jnp.float32),
            jax.ShapeDtypeStruct((_GRID, 1, _BLK), jnp.int32),
            jax.ShapeDtypeStruct((_GRID, 1, 1), jnp.float32),
        ],
    )(x, codewords)

    mean_e = jnp.sum(psum) / jnp.float32(_ROWS * _DIM)
    loss = mean_e + _COMMIT * mean_e
    return (q_st.reshape(in_shape),
            idx.reshape(in_shape[:-1]),
            loss)
